# Initial kernel scaffold; baseline (speedup 1.0000x reference)
import jax
import jax.numpy as jnp
from jax.experimental import pallas as pl


def kernel(user_feat, item_feat, by_src, by_time, pby_src, pby_time, W_user, W_item, Wg_u, Wg_i, i_te, i_te_k, u_te, u_te_k):
    raise NotImplementedError("write your pallas kernel here")


# speedup vs baseline: 1.2212x; 1.2212x over previous
import functools

import jax
import jax.numpy as jnp
from jax import lax
from jax.experimental import pallas as pl
from jax.experimental.pallas import tpu as pltpu
from jax.experimental.pallas import tpu_sc as plsc

N = 10000
L = 50
D = 128
LP = 56
NDST_PAD = 10240
NW = 32
DSTS_PER_TILE = NDST_PAD // NW
DSTS_PER_ITER = 4
ITERS = DSTS_PER_TILE // DSTS_PER_ITER
ROWS_PER_ITER = DSTS_PER_ITER * LP
BI_PREP = 200
BI_ATT = 200
INV_SQRT_D = 1.0 / (D ** 0.5)


def _prep_one(time_ref, src_ref, off):
    t = time_ref[...]
    src = src_ref[...]
    bi = t.shape[0]
    lane = lax.broadcasted_iota(jnp.int32, (1, L), 1)
    key = t * 64 + lane
    acc = jnp.zeros(t.shape, jnp.int32)
    for m in range(L):
        km = key[:, m:m + 1]
        acc = acc + lax.shift_right_arithmetic(km - key, 31)
    re = (L - 1) + acc
    idx_s = jnp.zeros(t.shape, jnp.int32)
    for l in range(L):
        idx_s = idx_s + jnp.where(re[:, l:l + 1] == lane, src[:, l:l + 1], 0)
    mx = jnp.max(t, axis=1, keepdims=True)
    cand = jnp.where(t == mx, lane, L)
    last = jnp.min(cand, axis=1, keepdims=True)
    idx_last = jnp.sum(jnp.where(lane == last, src, 0), axis=1, keepdims=True)
    return jnp.concatenate(
        [idx_s, idx_last, jnp.zeros((bi, LP - L - 1), jnp.int32)], axis=1) + off


def _prep_body(uf, itf, bsrc, btime, psrc, ptime, wu, wi,
               tu_o, ti_o, idxb_o, idxp_o):
    uh = lax.dot_general(uf[...], wu[...], (((1,), (1,)), ((), ())))
    ih = lax.dot_general(itf[...], wi[...], (((1,), (1,)), ((), ())))
    tu_o[...] = uh
    ti_o[...] = ih
    idxb_o[...] = _prep_one(btime, bsrc, 0)
    idxp_o[...] = _prep_one(ptime, psrc, N)


def _att_body(m_ref, dst_ref, feat_ref, te_ref, tek_ref, wg_ref, out_ref):
    last_em = m_ref[:, L, :]
    dst = dst_ref[...]
    bi = dst.shape[0]

    i_idx = lax.broadcasted_iota(jnp.int32, (8, 1, 16), 0)
    j_idx = lax.broadcasted_iota(jnp.int32, (8, 1, 16), 2)
    mask_e = (j_idx == i_idx).astype(jnp.float32)
    mask_1 = (j_idx == i_idx + 8).astype(jnp.float32)
    e_parts, e1_parts = [], []
    for g in range(bi // 8):
        mg = m_ref[g * 8:(g + 1) * 8, :, :]
        m2 = mg.reshape(8 * LP, D)
        dl = jnp.concatenate(
            [dst[g * 8:(g + 1) * 8, :], last_em[g * 8:(g + 1) * 8, :]], axis=0)
        s3 = lax.dot_general(m2, dl, (((1,), (1,)), ((), ())))
        s3 = s3.reshape(8, LP, 16)
        e_parts.append(jnp.sum(s3 * mask_e, axis=2))
        e1_parts.append(jnp.sum(s3 * mask_1, axis=2))
    e_raw = jnp.concatenate(e_parts, axis=0)
    e1_raw = jnp.concatenate(e1_parts, axis=0)

    p = lax.dot_general(dst, te_ref[...], (((1,), (1,)), ((), ())))
    lane = lax.broadcasted_iota(jnp.int32, (1, LP), 1)
    neg = jnp.where(lane >= L, -1e30, 0.0)
    e = (e_raw + p) * INV_SQRT_D + neg
    e1 = e1_raw * INV_SQRT_D + neg

    e = e - jnp.max(e, axis=1, keepdims=True)
    ex = jnp.exp(e)
    alpha = ex / jnp.sum(ex, axis=1, keepdims=True)
    e1 = e1 - jnp.max(e1, axis=1, keepdims=True)
    ex1 = jnp.exp(e1)
    alpha1 = ex1 / jnp.sum(ex1, axis=1, keepdims=True)

    h_long = jnp.zeros((bi, D), jnp.float32)
    h_short = jnp.zeros((bi, D), jnp.float32)
    for l in range(L):
        ml = m_ref[:, l, :]
        h_long = h_long + alpha[:, l:l + 1] * ml
        h_short = h_short + alpha1[:, l:l + 1] * ml
    h_long = h_long + lax.dot_general(alpha, tek_ref[...], (((1,), (0,)), ((), ())))

    h2 = jnp.concatenate([h_long, h_short], axis=1)
    now = lax.dot_general(h2, wg_ref[...], (((1,), (1,)), ((), ())))
    x = now + feat_ref[...]
    out_ref[...] = jnp.where(x > 0, x, jnp.exp(x) - 1.0)


def _prep_call(uf, itf, bsrc, btime, psrc, ptime, wu, wi):
    nb = N // BI_PREP
    row = lambda i: (i, 0)
    full = lambda i: (0, 0)
    return pl.pallas_call(
        _prep_body,
        grid=(nb,),
        in_specs=[
            pl.BlockSpec((BI_PREP, D), row),
            pl.BlockSpec((BI_PREP, D), row),
            pl.BlockSpec((BI_PREP, L), row),
            pl.BlockSpec((BI_PREP, L), row),
            pl.BlockSpec((BI_PREP, L), row),
            pl.BlockSpec((BI_PREP, L), row),
            pl.BlockSpec((D, D), full),
            pl.BlockSpec((D, D), full),
        ],
        out_specs=[
            pl.BlockSpec((BI_PREP, D), row),
            pl.BlockSpec((BI_PREP, D), row),
            pl.BlockSpec((BI_PREP, LP), row),
            pl.BlockSpec((BI_PREP, LP), row),
        ],
        out_shape=[
            jax.ShapeDtypeStruct((N, D), jnp.float32),
            jax.ShapeDtypeStruct((N, D), jnp.float32),
            jax.ShapeDtypeStruct((N, LP), jnp.int32),
            jax.ShapeDtypeStruct((N, LP), jnp.int32),
        ],
    )(uf, itf, bsrc, btime, psrc, ptime, wu, wi)


def _sc_gather(table, idx_flat):
    total = NDST_PAD * LP
    mesh = plsc.VectorSubcoreMesh(core_axis_name="c", subcore_axis_name="s")
    hw = ROWS_PER_ITER // 2

    @functools.partial(
        pl.kernel,
        mesh=mesh,
        out_type=jax.ShapeDtypeStruct((total, D), jnp.float32),
        scratch_types=[
            pltpu.VMEM((ROWS_PER_ITER,), jnp.int32),
            pltpu.VMEM((ROWS_PER_ITER, D), jnp.float32),
            pltpu.SemaphoreType.DMA,
        ],
    )
    def k(t_hbm, idx_hbm, out_hbm, idx_v, rows_v, sem):
        wid = lax.axis_index("s") * 2 + lax.axis_index("c")
        base = wid * (DSTS_PER_TILE * LP)

        def body(it, carry):
            off = base + it * ROWS_PER_ITER
            pltpu.sync_copy(idx_hbm.at[pl.ds(off, ROWS_PER_ITER)], idx_v)
            c1 = pltpu.async_copy(
                t_hbm.at[idx_v.at[pl.ds(0, hw)]], rows_v.at[pl.ds(0, hw)], sem)
            c2 = pltpu.async_copy(
                t_hbm.at[idx_v.at[pl.ds(hw, hw)]], rows_v.at[pl.ds(hw, hw)], sem)
            c1.wait()
            c2.wait()
            pltpu.sync_copy(rows_v, out_hbm.at[pl.ds(off, ROWS_PER_ITER)])
            return carry

        lax.fori_loop(0, ITERS, body, 0)

    return k(table, idx_flat)


def _att_call(m, dst, feat, te56, tek56, wg):
    nb = N // BI_ATT
    return pl.pallas_call(
        _att_body,
        grid=(nb,),
        in_specs=[
            pl.BlockSpec((BI_ATT, LP, D), lambda i: (i, 0, 0)),
            pl.BlockSpec((BI_ATT, D), lambda i: (i, 0)),
            pl.BlockSpec((BI_ATT, D), lambda i: (i, 0)),
            pl.BlockSpec((LP, D), lambda i: (0, 0)),
            pl.BlockSpec((LP, D), lambda i: (0, 0)),
            pl.BlockSpec((D, 2 * D), lambda i: (0, 0)),
        ],
        out_specs=pl.BlockSpec((BI_ATT, D), lambda i: (i, 0)),
        out_shape=jax.ShapeDtypeStruct((N, D), jnp.float32),
    )(m, dst, feat, te56, tek56, wg)


def kernel(user_feat, item_feat, by_src, by_time, pby_src, pby_time,
           W_user, W_item, Wg_u, Wg_i, i_te, i_te_k, u_te, u_te_k):
    by_src = by_src.astype(jnp.int32)
    by_time = by_time.astype(jnp.int32)
    pby_src = pby_src.astype(jnp.int32)
    pby_time = pby_time.astype(jnp.int32)

    (t_u, t_i, idx_by, idx_pby) = _prep_call(
        user_feat, item_feat, by_src, by_time, pby_src, pby_time,
        W_user, W_item)

    table = jnp.concatenate([t_u, t_i], axis=0)
    pad = jnp.zeros(((NDST_PAD - N) * LP,), jnp.int32)
    idx_by_f = jnp.concatenate([idx_by.reshape(-1), pad])
    idx_pby_f = jnp.concatenate([idx_pby.reshape(-1), pad + N])

    m_by = _sc_gather(table, idx_by_f).reshape(NDST_PAD, LP, D)
    m_pby = _sc_gather(table, idx_pby_f).reshape(NDST_PAD, LP, D)

    zpad = jnp.zeros((LP - L, D), jnp.float32)
    ite56 = jnp.concatenate([i_te, zpad], axis=0)
    itek56 = jnp.concatenate([i_te_k, zpad], axis=0)
    ute56 = jnp.concatenate([u_te, zpad], axis=0)
    utek56 = jnp.concatenate([u_te_k, zpad], axis=0)

    item_out = _att_call(m_by, t_i, item_feat, ite56, itek56, Wg_i)
    user_out = _att_call(m_pby, t_u, user_feat, ute56, utek56, Wg_u)
    return (user_out, item_out)
